# baseline (speedup 1.0000x reference)
import functools

import jax
import jax.numpy as jnp
from jax import lax
from jax.experimental import pallas as pl
from jax.experimental.pallas import tpu as pltpu
from jax.experimental.pallas import tpu_sc as plsc

B = 16384
D = 32
NC = 2
NS = 16
NW = NC * NS
BPW = B // NW
CHUNK = 128
NCH = BPW // CHUNK

_mesh = plsc.VectorSubcoreMesh(core_axis_name="c", subcore_axis_name="s")


@functools.partial(
    pl.kernel,
    mesh=_mesh,
    compiler_params=pltpu.CompilerParams(use_tc_tiling_on_sc=False),
    out_type=[jax.ShapeDtypeStruct((B, D), jnp.float32) for _ in range(4)],
    scratch_types=[
        pltpu.VMEM((NCH, CHUNK), jnp.int32),
        pltpu.VMEM((NCH, CHUNK), jnp.int32),
        pltpu.VMEM((BPW, D), jnp.float32),
        pltpu.VMEM((BPW, D), jnp.float32),
        pltpu.VMEM((BPW, D), jnp.float32),
        pltpu.VMEM((BPW, D), jnp.float32),
        pltpu.SemaphoreType.DMA,
    ],
)
def _sc_gather(u_idx_hbm, i_idx_hbm, ueg_hbm, ieg_hbm, uem_hbm, iem_hbm,
               oug, oig, oum, oim, idx_u, idx_i, bug, big, bum, bim, sem):
    wid = lax.axis_index("s") * NC + lax.axis_index("c")
    base = wid * BPW
    pltpu.sync_copy(u_idx_hbm.at[pl.ds(wid * NCH, NCH)], idx_u)
    pltpu.sync_copy(i_idx_hbm.at[pl.ds(wid * NCH, NCH)], idx_i)
    copies = []
    for tbl, buf, idx in ((ueg_hbm, bug, idx_u), (ieg_hbm, big, idx_i),
                          (uem_hbm, bum, idx_u), (iem_hbm, bim, idx_i)):
        for j in range(NCH):
            copies.append(pltpu.async_copy(
                tbl.at[idx.at[j]], buf.at[pl.ds(j * CHUNK, CHUNK)], sem))
    for c in copies:
        c.wait()
    pltpu.sync_copy(bug, oug.at[pl.ds(base, BPW)])
    pltpu.sync_copy(big, oig.at[pl.ds(base, BPW)])
    pltpu.sync_copy(bum, oum.at[pl.ds(base, BPW)])
    pltpu.sync_copy(bim, oim.at[pl.ds(base, BPW)])


TC_BLK = 2048


def _tc_body(uem, iem, ueg, ieg, w0, b0, w1, b1, w2, b2, w3, b3, fcw, fcb, out):
    h = jnp.concatenate([uem[...], iem[...]], axis=1)
    for w, b in ((w0, b0), (w1, b1), (w2, b2), (w3, b3)):
        h = lax.dot_general(h, w[...], (((1,), (1,)), ((), ())),
                            preferred_element_type=jnp.float32) + b[...]
        h = jnp.maximum(h, 0.0)
    gmf = ueg[...] * ieg[...]
    final = jnp.concatenate([gmf, h], axis=1)
    logit = jnp.sum(final * fcw[...], axis=1, keepdims=True) + fcb[...]
    out[...] = 1.0 / (1.0 + jnp.exp(-logit))


def _tc_dense(uem, iem, ueg, ieg, w0, b0, w1, b1, w2, b2, w3, b3, fcw, fcb):
    nblk = B // TC_BLK
    row_spec = pl.BlockSpec((TC_BLK, D), lambda i: (i, 0))

    def full(a):
        return pl.BlockSpec(a.shape, lambda i: tuple(0 for _ in a.shape))

    return pl.pallas_call(
        _tc_body,
        grid=(nblk,),
        in_specs=[row_spec, row_spec, row_spec, row_spec,
                  full(w0), full(b0), full(w1), full(b1),
                  full(w2), full(b2), full(w3), full(b3),
                  full(fcw), full(fcb)],
        out_specs=pl.BlockSpec((TC_BLK, 1), lambda i: (i, 0)),
        out_shape=jax.ShapeDtypeStruct((B, 1), jnp.float32),
    )(uem, iem, ueg, ieg, w0, b0, w1, b1, w2, b2, w3, b3, fcw, fcb)


def kernel(user_indices, item_indices, user_emb_gmf, item_emb_gmf,
           user_emb_mlp, item_emb_mlp, mlp_W0, mlp_b0, mlp_W1, mlp_b1,
           mlp_W2, mlp_b2, mlp_W3, mlp_b3, fc_W, fc_b):
    u2 = user_indices.astype(jnp.int32).reshape(NW * NCH, CHUNK)
    i2 = item_indices.astype(jnp.int32).reshape(NW * NCH, CHUNK)
    ueg, ieg, uem, iem = _sc_gather(
        u2, i2, user_emb_gmf, item_emb_gmf, user_emb_mlp, item_emb_mlp)
    return _tc_dense(
        uem, iem, ueg, ieg,
        mlp_W0, mlp_b0.reshape(1, -1), mlp_W1, mlp_b1.reshape(1, -1),
        mlp_W2, mlp_b2.reshape(1, -1), mlp_W3, mlp_b3.reshape(1, -1),
        fc_W, fc_b.reshape(1, -1))


# speedup vs baseline: 1.4415x; 1.4415x over previous
import functools

import jax
import jax.numpy as jnp
from jax import lax
from jax.experimental import pallas as pl
from jax.experimental.pallas import tpu as pltpu
from jax.experimental.pallas import tpu_sc as plsc

B = 16384
D = 32
NC = 2
NS = 16
NW = NC * NS
BPW = B // NW

_mesh = plsc.VectorSubcoreMesh(core_axis_name="c", subcore_axis_name="s")


@functools.partial(
    pl.kernel,
    mesh=_mesh,
    out_type=jax.ShapeDtypeStruct((B, 4 * D), jnp.float32),
    scratch_types=[
        pltpu.VMEM((BPW,), jnp.int32),
        pltpu.VMEM((BPW,), jnp.int32),
        pltpu.VMEM((BPW, 4 * D), jnp.float32),
        pltpu.SemaphoreType.DMA,
    ],
)
def _sc_gather(u_hbm, i_hbm, ueg_hbm, ieg_hbm, uem_hbm, iem_hbm,
               out_hbm, su_v, si_v, out_v, sem):
    wid = lax.axis_index("s") * NC + lax.axis_index("c")
    base = wid * BPW
    pltpu.sync_copy(u_hbm.at[pl.ds(base, BPW)], su_v)
    pltpu.sync_copy(i_hbm.at[pl.ds(base, BPW)], si_v)

    def body(g, carry):
        uvec = su_v[pl.ds(g * 16, 16)]
        ivec = si_v[pl.ds(g * 16, 16)]
        for l in range(16):
            j = g * 16 + l
            ru = uvec[l]
            ri = ivec[l]
            pltpu.async_copy(ueg_hbm.at[ru], out_v.at[j, pl.ds(0 * D, D)], sem)
            pltpu.async_copy(ieg_hbm.at[ri], out_v.at[j, pl.ds(1 * D, D)], sem)
            pltpu.async_copy(uem_hbm.at[ru], out_v.at[j, pl.ds(2 * D, D)], sem)
            pltpu.async_copy(iem_hbm.at[ri], out_v.at[j, pl.ds(3 * D, D)], sem)
        return carry

    lax.fori_loop(0, BPW // 16, body, 0)
    pltpu.make_async_copy(out_hbm.at[pl.ds(base, BPW)], out_v, sem).wait()
    pltpu.sync_copy(out_v, out_hbm.at[pl.ds(base, BPW)])


TC_BLK = 2048


def _tc_body(x, w0, b0, w1, b1, w2, b2, w3, b3, fcw, fcb, out):
    xv = x[...]
    gmf = xv[:, 0 * D:1 * D] * xv[:, 1 * D:2 * D]
    h = xv[:, 2 * D:4 * D]
    for w, b in ((w0, b0), (w1, b1), (w2, b2), (w3, b3)):
        h = lax.dot_general(h, w[...], (((1,), (1,)), ((), ())),
                            preferred_element_type=jnp.float32) + b[...]
        h = jnp.maximum(h, 0.0)
    final = jnp.concatenate([gmf, h], axis=1)
    logit = jnp.sum(final * fcw[...], axis=1, keepdims=True) + fcb[...]
    out[...] = 1.0 / (1.0 + jnp.exp(-logit))


def _tc_dense(x, w0, b0, w1, b1, w2, b2, w3, b3, fcw, fcb):
    nblk = B // TC_BLK

    def full(a):
        return pl.BlockSpec(a.shape, lambda i: tuple(0 for _ in a.shape))

    return pl.pallas_call(
        _tc_body,
        grid=(nblk,),
        in_specs=[pl.BlockSpec((TC_BLK, 4 * D), lambda i: (i, 0)),
                  full(w0), full(b0), full(w1), full(b1),
                  full(w2), full(b2), full(w3), full(b3),
                  full(fcw), full(fcb)],
        out_specs=pl.BlockSpec((TC_BLK, 1), lambda i: (i, 0)),
        out_shape=jax.ShapeDtypeStruct((B, 1), jnp.float32),
    )(x, w0, b0, w1, b1, w2, b2, w3, b3, fcw, fcb)


def kernel(user_indices, item_indices, user_emb_gmf, item_emb_gmf,
           user_emb_mlp, item_emb_mlp, mlp_W0, mlp_b0, mlp_W1, mlp_b1,
           mlp_W2, mlp_b2, mlp_W3, mlp_b3, fc_W, fc_b):
    ui = user_indices.astype(jnp.int32)
    ii = item_indices.astype(jnp.int32)
    packed = _sc_gather(ui, ii, user_emb_gmf, item_emb_gmf,
                        user_emb_mlp, item_emb_mlp)
    return _tc_dense(
        packed,
        mlp_W0, mlp_b0.reshape(1, -1), mlp_W1, mlp_b1.reshape(1, -1),
        mlp_W2, mlp_b2.reshape(1, -1), mlp_W3, mlp_b3.reshape(1, -1),
        fc_W, fc_b.reshape(1, -1))


# speedup vs baseline: 4.3052x; 2.9867x over previous
import functools

import jax
import jax.numpy as jnp
from jax import lax
from jax.experimental import pallas as pl
from jax.experimental.pallas import tpu as pltpu
from jax.experimental.pallas import tpu_sc as plsc

B = 16384
D = 32
V = 1000000
VTAIL = V - (V // 128) * 128
VMAIN = V - VTAIL
NC = 2
NS = 16
NW = NC * NS
VPW = V // NW
CV = 1152
NCHK = 28
LCAP = 2080
HCAP = 160
RING = 8

_mesh = plsc.VectorSubcoreMesh(core_axis_name="c", subcore_axis_name="s")


@functools.partial(
    pl.kernel,
    mesh=_mesh,
    compiler_params=pltpu.CompilerParams(needs_layout_passes=False),
    out_type=jax.ShapeDtypeStruct((B, 4 * D), jnp.float32),
    scratch_types=[
        pltpu.VMEM((B,), jnp.int32),
        pltpu.VMEM((LCAP,), jnp.int32),
        pltpu.VMEM((LCAP,), jnp.int32),
        pltpu.VMEM((HCAP,), jnp.int32),
        pltpu.VMEM((HCAP,), jnp.int32),
        pltpu.VMEM((D, CV), jnp.float32),
        pltpu.VMEM((D, CV), jnp.float32),
        pltpu.VMEM((D, VTAIL), jnp.float32),
        pltpu.VMEM((D, VTAIL), jnp.float32),
        pltpu.VMEM((RING, D), jnp.float32),
        pltpu.VMEM((RING, D), jnp.float32),
        pltpu.SemaphoreType.DMA,
    ],
)
def _sc_gather(u_hbm, i_hbm, ueg_t, ieg_t, uem_t, iem_t,
               ueg_tail, ieg_tail, uem_tail, iem_tail,
               out_hbm, idx_v, lidx, lpos, hrloc, hpos,
               bufa, bufb, taila, tailb, stga, stgb, sem):
    cid = lax.axis_index("c")
    sid = lax.axis_index("s")
    wid = sid * NC + cid
    lo = wid * VPW
    hi = lo + VPW
    cbase0 = (lo // 128) * 128
    lane = lax.iota(jnp.int32, 16)

    def run_pass(src_idx_hbm, tbl_a, tbl_b, tail_a, tail_b, col_a, col_b):
        pltpu.sync_copy(src_idx_hbm, idx_v)

        def filt(g, p):
            v = idx_v[pl.ds(g * 16, 16)]
            pos = g * 16 + lane
            m = (v >= lo) & (v < hi)
            plsc.store_compressed(lidx.at[pl.ds(p, 16)], v, mask=m)
            plsc.store_compressed(lpos.at[pl.ds(p, 16)], pos, mask=m)
            return p + plsc.all_reduce_population_count(m)[0]

        n = lax.fori_loop(0, B // 16, filt, jnp.int32(0))
        nvec = (n + 15) // 16

        def extract_hits(nh, buf_a, buf_b):
            def ext(e, carry):
                slot = lax.rem(e, RING)
                rloc = hrloc[pl.ds(e, 16)][0]
                pos = hpos[pl.ds(e, 16)][0]
                rsp = jnp.full((16,), 0, jnp.int32) + rloc
                ssp = jnp.full((16,), 0, jnp.int32) + slot

                @pl.when(e >= RING)
                def _():
                    pltpu.make_async_copy(
                        stga.at[slot], out_hbm.at[0, pl.ds(col_a, D)],
                        sem).wait()
                    pltpu.make_async_copy(
                        stgb.at[slot], out_hbm.at[0, pl.ds(col_b, D)],
                        sem).wait()

                plsc.store_scatter(stga, [ssp, lane],
                                   plsc.load_gather(buf_a, [lane, rsp]))
                plsc.store_scatter(stga, [ssp, lane + 16],
                                   plsc.load_gather(buf_a, [lane + 16, rsp]))
                plsc.store_scatter(stgb, [ssp, lane],
                                   plsc.load_gather(buf_b, [lane, rsp]))
                plsc.store_scatter(stgb, [ssp, lane + 16],
                                   plsc.load_gather(buf_b, [lane + 16, rsp]))
                pltpu.async_copy(stga.at[slot],
                                 out_hbm.at[pos, pl.ds(col_a, D)], sem)
                pltpu.async_copy(stgb.at[slot],
                                 out_hbm.at[pos, pl.ds(col_b, D)], sem)
                return carry

            lax.fori_loop(0, nh, ext, jnp.int32(0))

            def drain(e, carry):
                pltpu.make_async_copy(
                    stga.at[0], out_hbm.at[0, pl.ds(col_a, D)], sem).wait()
                pltpu.make_async_copy(
                    stgb.at[0], out_hbm.at[0, pl.ds(col_b, D)], sem).wait()
                return carry

            lax.fori_loop(0, jnp.minimum(nh, RING), drain, jnp.int32(0))

        def chunk(m, carry):
            cbase = jnp.minimum(cbase0 + m * CV, VMAIN - CV)
            cbase = pl.multiple_of(cbase, 128)
            pltpu.sync_copy(tbl_a.at[:, pl.ds(cbase, CV)], bufa)
            pltpu.sync_copy(tbl_b.at[:, pl.ds(cbase, CV)], bufb)

            def scan(g, q):
                v = lidx[pl.ds(g * 16, 16)]
                p = lpos[pl.ds(g * 16, 16)]
                rl = v - cbase
                hm = ((rl >= 0) & (rl < CV) & (v < VMAIN)
                      & (g * 16 + lane < n))
                plsc.store_compressed(hrloc.at[pl.ds(q, 16)], rl, mask=hm)
                plsc.store_compressed(hpos.at[pl.ds(q, 16)], p, mask=hm)
                return q + plsc.all_reduce_population_count(hm)[0]

            nh = lax.fori_loop(0, nvec, scan, jnp.int32(0))
            extract_hits(nh, bufa, bufb)
            return carry

        lax.fori_loop(0, NCHK, chunk, jnp.int32(0))

        pltpu.sync_copy(tail_a, taila)
        pltpu.sync_copy(tail_b, tailb)

        def tscan(g, q):
            v = lidx[pl.ds(g * 16, 16)]
            p = lpos[pl.ds(g * 16, 16)]
            rl = v - VMAIN
            hm = (rl >= 0) & (g * 16 + lane < n)
            plsc.store_compressed(hrloc.at[pl.ds(q, 16)], rl, mask=hm)
            plsc.store_compressed(hpos.at[pl.ds(q, 16)], p, mask=hm)
            return q + plsc.all_reduce_population_count(hm)[0]

        nt = lax.fori_loop(0, nvec, tscan, jnp.int32(0))
        extract_hits(nt, taila, tailb)

    run_pass(u_hbm, ueg_t, uem_t, ueg_tail, uem_tail, 0 * D, 2 * D)
    run_pass(i_hbm, ieg_t, iem_t, ieg_tail, iem_tail, 1 * D, 3 * D)


TC_BLK = 2048


def _tc_body(x, w0, b0, w1, b1, w2, b2, w3, b3, fcw, fcb, out):
    xv = x[...]
    gmf = xv[:, 0 * D:1 * D] * xv[:, 1 * D:2 * D]
    h = xv[:, 2 * D:4 * D]
    for w, b in ((w0, b0), (w1, b1), (w2, b2), (w3, b3)):
        h = lax.dot_general(h, w[...], (((1,), (1,)), ((), ())),
                            preferred_element_type=jnp.float32) + b[...]
        h = jnp.maximum(h, 0.0)
    final = jnp.concatenate([gmf, h], axis=1)
    logit = jnp.sum(final * fcw[...], axis=1, keepdims=True) + fcb[...]
    out[...] = 1.0 / (1.0 + jnp.exp(-logit))


def _tc_dense(x, w0, b0, w1, b1, w2, b2, w3, b3, fcw, fcb):
    nblk = B // TC_BLK

    def full(a):
        return pl.BlockSpec(a.shape, lambda i: tuple(0 for _ in a.shape))

    return pl.pallas_call(
        _tc_body,
        grid=(nblk,),
        in_specs=[pl.BlockSpec((TC_BLK, 4 * D), lambda i: (i, 0)),
                  full(w0), full(b0), full(w1), full(b1),
                  full(w2), full(b2), full(w3), full(b3),
                  full(fcw), full(fcb)],
        out_specs=pl.BlockSpec((TC_BLK, 1), lambda i: (i, 0)),
        out_shape=jax.ShapeDtypeStruct((B, 1), jnp.float32),
    )(x, w0, b0, w1, b1, w2, b2, w3, b3, fcw, fcb)


def kernel(user_indices, item_indices, user_emb_gmf, item_emb_gmf,
           user_emb_mlp, item_emb_mlp, mlp_W0, mlp_b0, mlp_W1, mlp_b1,
           mlp_W2, mlp_b2, mlp_W3, mlp_b3, fc_W, fc_b):
    ui = user_indices.astype(jnp.int32)
    ii = item_indices.astype(jnp.int32)
    tails = [t[VMAIN:].T for t in (user_emb_gmf, item_emb_gmf,
                                   user_emb_mlp, item_emb_mlp)]
    packed = _sc_gather(ui, ii, user_emb_gmf.T, item_emb_gmf.T,
                        user_emb_mlp.T, item_emb_mlp.T, *tails)
    return _tc_dense(
        packed,
        mlp_W0, mlp_b0.reshape(1, -1), mlp_W1, mlp_b1.reshape(1, -1),
        mlp_W2, mlp_b2.reshape(1, -1), mlp_W3, mlp_b3.reshape(1, -1),
        fc_W, fc_b.reshape(1, -1))


# speedup vs baseline: 5.3884x; 1.2516x over previous
import functools

import jax
import jax.numpy as jnp
from jax import lax
from jax.experimental import pallas as pl
from jax.experimental.pallas import tpu as pltpu
from jax.experimental.pallas import tpu_sc as plsc

B = 16384
D = 32
V = 1000000
VTAIL = V - (V // 128) * 128
VMAIN = V - VTAIL
NC = 2
NS = 16
NW = NC * NS
VPW = V // NW
CV = 640
NCHK = 50
LCAP = 2080
HCAP = 160
RING = 8

_mesh = plsc.VectorSubcoreMesh(core_axis_name="c", subcore_axis_name="s")


@functools.partial(
    pl.kernel,
    mesh=_mesh,
    compiler_params=pltpu.CompilerParams(needs_layout_passes=False),
    out_type=jax.ShapeDtypeStruct((B, 4 * D), jnp.float32),
    scratch_types=[
        pltpu.VMEM((B,), jnp.int32),
        pltpu.VMEM((LCAP,), jnp.int32),
        pltpu.VMEM((LCAP,), jnp.int32),
        pltpu.VMEM((HCAP,), jnp.int32),
        pltpu.VMEM((HCAP,), jnp.int32),
        pltpu.VMEM((D, CV), jnp.float32),
        pltpu.VMEM((D, CV), jnp.float32),
        pltpu.VMEM((D, CV), jnp.float32),
        pltpu.VMEM((D, CV), jnp.float32),
        pltpu.VMEM((D, VTAIL), jnp.float32),
        pltpu.VMEM((D, VTAIL), jnp.float32),
        pltpu.VMEM((RING, D), jnp.float32),
        pltpu.VMEM((RING, D), jnp.float32),
        pltpu.SemaphoreType.DMA,
        pltpu.SemaphoreType.DMA,
    ],
)
def _sc_gather(u_hbm, i_hbm, ueg_t, ieg_t, uem_t, iem_t,
               ueg_tail, ieg_tail, uem_tail, iem_tail,
               out_hbm, idx_v, lidx, lpos, hrloc, hpos,
               bufa0, bufb0, bufa1, bufb1, taila, tailb, stga, stgb,
               sem, fsem):
    cid = lax.axis_index("c")
    sid = lax.axis_index("s")
    wid = sid * NC + cid
    lo = wid * VPW
    hi = lo + VPW
    cbase0 = (lo // 128) * 128
    lane = lax.iota(jnp.int32, 16)

    def run_pass(src_idx_hbm, tbl_a, tbl_b, tail_a, tail_b, col_a, col_b):
        pltpu.sync_copy(src_idx_hbm, idx_v)

        def filt(g, p):
            v = idx_v[pl.ds(g * 16, 16)]
            pos = g * 16 + lane
            m = (v >= lo) & (v < hi)
            plsc.store_compressed(lidx.at[pl.ds(p, 16)], v, mask=m)
            plsc.store_compressed(lpos.at[pl.ds(p, 16)], pos, mask=m)
            return p + plsc.all_reduce_population_count(m)[0]

        n = lax.fori_loop(0, B // 16, filt, jnp.int32(0))
        nvec = (n + 15) // 16

        def extract_hits(nh, buf_a, buf_b):
            def ext(e, carry):
                slot = lax.rem(e, RING)
                rloc = hrloc[pl.ds(e, 16)][0]
                pos = hpos[pl.ds(e, 16)][0]
                rsp = jnp.full((16,), 0, jnp.int32) + rloc
                ssp = jnp.full((16,), 0, jnp.int32) + slot

                @pl.when(e >= RING)
                def _():
                    pltpu.make_async_copy(
                        stga.at[slot], out_hbm.at[0, pl.ds(col_a, D)],
                        sem).wait()
                    pltpu.make_async_copy(
                        stgb.at[slot], out_hbm.at[0, pl.ds(col_b, D)],
                        sem).wait()

                plsc.store_scatter(stga, [ssp, lane],
                                   plsc.load_gather(buf_a, [lane, rsp]))
                plsc.store_scatter(stga, [ssp, lane + 16],
                                   plsc.load_gather(buf_a, [lane + 16, rsp]))
                plsc.store_scatter(stgb, [ssp, lane],
                                   plsc.load_gather(buf_b, [lane, rsp]))
                plsc.store_scatter(stgb, [ssp, lane + 16],
                                   plsc.load_gather(buf_b, [lane + 16, rsp]))
                pltpu.async_copy(stga.at[slot],
                                 out_hbm.at[pos, pl.ds(col_a, D)], sem)
                pltpu.async_copy(stgb.at[slot],
                                 out_hbm.at[pos, pl.ds(col_b, D)], sem)
                return carry

            lax.fori_loop(0, nh, ext, jnp.int32(0))

            def drain(e, carry):
                pltpu.make_async_copy(
                    stga.at[0], out_hbm.at[0, pl.ds(col_a, D)], sem).wait()
                pltpu.make_async_copy(
                    stgb.at[0], out_hbm.at[0, pl.ds(col_b, D)], sem).wait()
                return carry

            lax.fori_loop(0, jnp.minimum(nh, RING), drain, jnp.int32(0))

        def cb(m):
            c = jnp.minimum(cbase0 + m * CV, VMAIN - CV)
            return pl.multiple_of(c, 128)

        def fetch(m, buf_a, buf_b):
            c = cb(m)
            pltpu.async_copy(tbl_a.at[:, pl.ds(c, CV)], buf_a, fsem)
            pltpu.async_copy(tbl_b.at[:, pl.ds(c, CV)], buf_b, fsem)

        def fwait(buf_a, buf_b):
            c = pl.multiple_of(jnp.int32(0), 128)
            pltpu.make_async_copy(
                tbl_a.at[:, pl.ds(c, CV)], buf_a, fsem).wait()
            pltpu.make_async_copy(
                tbl_b.at[:, pl.ds(c, CV)], buf_b, fsem).wait()

        def process(m, buf_a, buf_b):
            cbase = cb(m)

            def scan(g, q):
                v = lidx[pl.ds(g * 16, 16)]
                p = lpos[pl.ds(g * 16, 16)]
                rl = v - cbase
                hm = ((rl >= 0) & (rl < CV) & (v < VMAIN)
                      & (g * 16 + lane < n))
                plsc.store_compressed(hrloc.at[pl.ds(q, 16)], rl, mask=hm)
                plsc.store_compressed(hpos.at[pl.ds(q, 16)], p, mask=hm)
                return q + plsc.all_reduce_population_count(hm)[0]

            nh = lax.fori_loop(0, nvec, scan, jnp.int32(0))
            extract_hits(nh, buf_a, buf_b)

        fetch(jnp.int32(0), bufa0, bufb0)

        def chunk2(k, carry):
            m0 = k * 2
            fwait(bufa0, bufb0)
            fetch(m0 + 1, bufa1, bufb1)
            process(m0, bufa0, bufb0)
            fwait(bufa1, bufb1)
            fetch(m0 + 2, bufa0, bufb0)
            process(m0 + 1, bufa1, bufb1)
            return carry

        lax.fori_loop(0, NCHK // 2, chunk2, jnp.int32(0))
        fwait(bufa0, bufb0)

        pltpu.sync_copy(tail_a, taila)
        pltpu.sync_copy(tail_b, tailb)

        def tscan(g, q):
            v = lidx[pl.ds(g * 16, 16)]
            p = lpos[pl.ds(g * 16, 16)]
            rl = v - VMAIN
            hm = (rl >= 0) & (g * 16 + lane < n)
            plsc.store_compressed(hrloc.at[pl.ds(q, 16)], rl, mask=hm)
            plsc.store_compressed(hpos.at[pl.ds(q, 16)], p, mask=hm)
            return q + plsc.all_reduce_population_count(hm)[0]

        nt = lax.fori_loop(0, nvec, tscan, jnp.int32(0))
        extract_hits(nt, taila, tailb)

    run_pass(u_hbm, ueg_t, uem_t, ueg_tail, uem_tail, 0 * D, 2 * D)
    run_pass(i_hbm, ieg_t, iem_t, ieg_tail, iem_tail, 1 * D, 3 * D)


TC_BLK = 2048


def _tc_body(x, w0, b0, w1, b1, w2, b2, w3, b3, fcw, fcb, out):
    xv = x[...]
    gmf = xv[:, 0 * D:1 * D] * xv[:, 1 * D:2 * D]
    h = xv[:, 2 * D:4 * D]
    for w, b in ((w0, b0), (w1, b1), (w2, b2), (w3, b3)):
        h = lax.dot_general(h, w[...], (((1,), (1,)), ((), ())),
                            preferred_element_type=jnp.float32) + b[...]
        h = jnp.maximum(h, 0.0)
    final = jnp.concatenate([gmf, h], axis=1)
    logit = jnp.sum(final * fcw[...], axis=1, keepdims=True) + fcb[...]
    out[...] = 1.0 / (1.0 + jnp.exp(-logit))


def _tc_dense(x, w0, b0, w1, b1, w2, b2, w3, b3, fcw, fcb):
    nblk = B // TC_BLK

    def full(a):
        return pl.BlockSpec(a.shape, lambda i: tuple(0 for _ in a.shape))

    return pl.pallas_call(
        _tc_body,
        grid=(nblk,),
        in_specs=[pl.BlockSpec((TC_BLK, 4 * D), lambda i: (i, 0)),
                  full(w0), full(b0), full(w1), full(b1),
                  full(w2), full(b2), full(w3), full(b3),
                  full(fcw), full(fcb)],
        out_specs=pl.BlockSpec((TC_BLK, 1), lambda i: (i, 0)),
        out_shape=jax.ShapeDtypeStruct((B, 1), jnp.float32),
    )(x, w0, b0, w1, b1, w2, b2, w3, b3, fcw, fcb)


def kernel(user_indices, item_indices, user_emb_gmf, item_emb_gmf,
           user_emb_mlp, item_emb_mlp, mlp_W0, mlp_b0, mlp_W1, mlp_b1,
           mlp_W2, mlp_b2, mlp_W3, mlp_b3, fc_W, fc_b):
    ui = user_indices.astype(jnp.int32)
    ii = item_indices.astype(jnp.int32)
    tails = [t[VMAIN:].T for t in (user_emb_gmf, item_emb_gmf,
                                   user_emb_mlp, item_emb_mlp)]
    packed = _sc_gather(ui, ii, user_emb_gmf.T, item_emb_gmf.T,
                        user_emb_mlp.T, item_emb_mlp.T, *tails)
    return _tc_dense(
        packed,
        mlp_W0, mlp_b0.reshape(1, -1), mlp_W1, mlp_b1.reshape(1, -1),
        mlp_W2, mlp_b2.reshape(1, -1), mlp_W3, mlp_b3.reshape(1, -1),
        fc_W, fc_b.reshape(1, -1))


# speedup vs baseline: 5.5957x; 1.0385x over previous
import functools

import jax
import jax.numpy as jnp
from jax import lax
from jax.experimental import pallas as pl
from jax.experimental.pallas import tpu as pltpu
from jax.experimental.pallas import tpu_sc as plsc

B = 16384
D = 32
V = 1000000
VTAIL = V - (V // 128) * 128
VMAIN = V - VTAIL
NC = 2
NS = 16
NW = NC * NS
VPW = V // NW
CV = 768
NCHK = 42
LCAP = 2080
HCAP = 160
RING = 8

_mesh = plsc.VectorSubcoreMesh(core_axis_name="c", subcore_axis_name="s")


@functools.partial(
    pl.kernel,
    mesh=_mesh,
    compiler_params=pltpu.CompilerParams(needs_layout_passes=False),
    out_type=jax.ShapeDtypeStruct((B, 4 * D), jnp.float32),
    scratch_types=[
        pltpu.VMEM((B,), jnp.int32),
        pltpu.VMEM((LCAP,), jnp.int32),
        pltpu.VMEM((LCAP,), jnp.int32),
        pltpu.VMEM((HCAP,), jnp.int32),
        pltpu.VMEM((HCAP,), jnp.int32),
        pltpu.VMEM((D, CV), jnp.float32),
        pltpu.VMEM((D, CV), jnp.float32),
        pltpu.VMEM((D, CV), jnp.float32),
        pltpu.VMEM((D, CV), jnp.float32),
        pltpu.VMEM((D, VTAIL), jnp.float32),
        pltpu.VMEM((D, VTAIL), jnp.float32),
        pltpu.VMEM((RING, D), jnp.float32),
        pltpu.VMEM((RING, D), jnp.float32),
        pltpu.SemaphoreType.DMA,
        pltpu.SemaphoreType.DMA,
    ],
)
def _sc_gather(u_hbm, i_hbm, ueg_t, ieg_t, uem_t, iem_t,
               ueg_tail, ieg_tail, uem_tail, iem_tail,
               out_hbm, idx_v, lidx, lpos, hrloc, hpos,
               bufa0, bufb0, bufa1, bufb1, taila, tailb, stga, stgb,
               sem, fsem):
    cid = lax.axis_index("c")
    sid = lax.axis_index("s")
    wid = sid * NC + cid
    lo = wid * VPW
    hi = lo + VPW
    cbase0 = (lo // 128) * 128
    lane = lax.iota(jnp.int32, 16)

    def run_pass(src_idx_hbm, tbl_a, tbl_b, tail_a, tail_b, col_a, col_b):
        c0 = pl.multiple_of(cbase0, 128)
        pltpu.async_copy(tbl_a.at[:, pl.ds(c0, CV)], bufa0, fsem)
        pltpu.async_copy(tbl_b.at[:, pl.ds(c0, CV)], bufb0, fsem)
        pltpu.sync_copy(src_idx_hbm, idx_v)

        def filt(g, p):
            v = idx_v[pl.ds(g * 16, 16)]
            pos = g * 16 + lane
            m = (v >= lo) & (v < hi)
            plsc.store_compressed(lidx.at[pl.ds(p, 16)], v, mask=m)
            plsc.store_compressed(lpos.at[pl.ds(p, 16)], pos, mask=m)
            return p + plsc.all_reduce_population_count(m)[0]

        n = lax.fori_loop(0, B // 16, filt, jnp.int32(0))
        nvec = (n + 15) // 16

        def extract_hits(nh, buf_a, buf_b):
            def ext(e, carry):
                slot = lax.rem(e, RING)
                rloc = hrloc[pl.ds(e, 16)][0]
                pos = hpos[pl.ds(e, 16)][0]
                rsp = jnp.full((16,), 0, jnp.int32) + rloc
                ssp = jnp.full((16,), 0, jnp.int32) + slot

                @pl.when(e >= RING)
                def _():
                    pltpu.make_async_copy(
                        stga.at[slot], out_hbm.at[0, pl.ds(col_a, D)],
                        sem).wait()
                    pltpu.make_async_copy(
                        stgb.at[slot], out_hbm.at[0, pl.ds(col_b, D)],
                        sem).wait()

                plsc.store_scatter(stga, [ssp, lane],
                                   plsc.load_gather(buf_a, [lane, rsp]))
                plsc.store_scatter(stga, [ssp, lane + 16],
                                   plsc.load_gather(buf_a, [lane + 16, rsp]))
                plsc.store_scatter(stgb, [ssp, lane],
                                   plsc.load_gather(buf_b, [lane, rsp]))
                plsc.store_scatter(stgb, [ssp, lane + 16],
                                   plsc.load_gather(buf_b, [lane + 16, rsp]))
                pltpu.async_copy(stga.at[slot],
                                 out_hbm.at[pos, pl.ds(col_a, D)], sem)
                pltpu.async_copy(stgb.at[slot],
                                 out_hbm.at[pos, pl.ds(col_b, D)], sem)
                return carry

            lax.fori_loop(0, nh, ext, jnp.int32(0))

            def drain(e, carry):
                pltpu.make_async_copy(
                    stga.at[0], out_hbm.at[0, pl.ds(col_a, D)], sem).wait()
                pltpu.make_async_copy(
                    stgb.at[0], out_hbm.at[0, pl.ds(col_b, D)], sem).wait()
                return carry

            lax.fori_loop(0, jnp.minimum(nh, RING), drain, jnp.int32(0))

        def cb(m):
            c = jnp.minimum(cbase0 + m * CV, VMAIN - CV)
            return pl.multiple_of(c, 128)

        def fetch(m, buf_a, buf_b):
            c = cb(m)
            pltpu.async_copy(tbl_a.at[:, pl.ds(c, CV)], buf_a, fsem)
            pltpu.async_copy(tbl_b.at[:, pl.ds(c, CV)], buf_b, fsem)

        def fwait(buf_a, buf_b):
            c = pl.multiple_of(jnp.int32(0), 128)
            pltpu.make_async_copy(
                tbl_a.at[:, pl.ds(c, CV)], buf_a, fsem).wait()
            pltpu.make_async_copy(
                tbl_b.at[:, pl.ds(c, CV)], buf_b, fsem).wait()

        def process(m, buf_a, buf_b):
            cbase = cb(m)

            def scan(g, q):
                v = lidx[pl.ds(g * 16, 16)]
                p = lpos[pl.ds(g * 16, 16)]
                rl = v - cbase
                hm = ((rl >= 0) & (rl < CV) & (v < VMAIN)
                      & (g * 16 + lane < n))
                plsc.store_compressed(hrloc.at[pl.ds(q, 16)], rl, mask=hm)
                plsc.store_compressed(hpos.at[pl.ds(q, 16)], p, mask=hm)
                return q + plsc.all_reduce_population_count(hm)[0]

            nh = lax.fori_loop(0, nvec, scan, jnp.int32(0))
            extract_hits(nh, buf_a, buf_b)

        def chunk2(k, carry):
            m0 = k * 2
            fwait(bufa0, bufb0)
            fetch(m0 + 1, bufa1, bufb1)
            process(m0, bufa0, bufb0)
            fwait(bufa1, bufb1)
            fetch(m0 + 2, bufa0, bufb0)
            process(m0 + 1, bufa1, bufb1)
            return carry

        lax.fori_loop(0, NCHK // 2, chunk2, jnp.int32(0))
        fwait(bufa0, bufb0)

        pltpu.sync_copy(tail_a, taila)
        pltpu.sync_copy(tail_b, tailb)

        def tscan(g, q):
            v = lidx[pl.ds(g * 16, 16)]
            p = lpos[pl.ds(g * 16, 16)]
            rl = v - VMAIN
            hm = (rl >= 0) & (g * 16 + lane < n)
            plsc.store_compressed(hrloc.at[pl.ds(q, 16)], rl, mask=hm)
            plsc.store_compressed(hpos.at[pl.ds(q, 16)], p, mask=hm)
            return q + plsc.all_reduce_population_count(hm)[0]

        nt = lax.fori_loop(0, nvec, tscan, jnp.int32(0))
        extract_hits(nt, taila, tailb)

    run_pass(u_hbm, ueg_t, uem_t, ueg_tail, uem_tail, 0 * D, 2 * D)
    run_pass(i_hbm, ieg_t, iem_t, ieg_tail, iem_tail, 1 * D, 3 * D)


TC_BLK = 2048


def _tc_body(x, w0, b0, w1, b1, w2, b2, w3, b3, fcw, fcb, out):
    xv = x[...]
    gmf = xv[:, 0 * D:1 * D] * xv[:, 1 * D:2 * D]
    h = xv[:, 2 * D:4 * D]
    for w, b in ((w0, b0), (w1, b1), (w2, b2), (w3, b3)):
        h = lax.dot_general(h, w[...], (((1,), (1,)), ((), ())),
                            preferred_element_type=jnp.float32) + b[...]
        h = jnp.maximum(h, 0.0)
    final = jnp.concatenate([gmf, h], axis=1)
    logit = jnp.sum(final * fcw[...], axis=1, keepdims=True) + fcb[...]
    out[...] = 1.0 / (1.0 + jnp.exp(-logit))


def _tc_dense(x, w0, b0, w1, b1, w2, b2, w3, b3, fcw, fcb):
    nblk = B // TC_BLK

    def full(a):
        return pl.BlockSpec(a.shape, lambda i: tuple(0 for _ in a.shape))

    return pl.pallas_call(
        _tc_body,
        grid=(nblk,),
        in_specs=[pl.BlockSpec((TC_BLK, 4 * D), lambda i: (i, 0)),
                  full(w0), full(b0), full(w1), full(b1),
                  full(w2), full(b2), full(w3), full(b3),
                  full(fcw), full(fcb)],
        out_specs=pl.BlockSpec((TC_BLK, 1), lambda i: (i, 0)),
        out_shape=jax.ShapeDtypeStruct((B, 1), jnp.float32),
    )(x, w0, b0, w1, b1, w2, b2, w3, b3, fcw, fcb)


def kernel(user_indices, item_indices, user_emb_gmf, item_emb_gmf,
           user_emb_mlp, item_emb_mlp, mlp_W0, mlp_b0, mlp_W1, mlp_b1,
           mlp_W2, mlp_b2, mlp_W3, mlp_b3, fc_W, fc_b):
    ui = user_indices.astype(jnp.int32)
    ii = item_indices.astype(jnp.int32)
    tails = [t[VMAIN:].T for t in (user_emb_gmf, item_emb_gmf,
                                   user_emb_mlp, item_emb_mlp)]
    packed = _sc_gather(ui, ii, user_emb_gmf.T, item_emb_gmf.T,
                        user_emb_mlp.T, item_emb_mlp.T, *tails)
    return _tc_dense(
        packed,
        mlp_W0, mlp_b0.reshape(1, -1), mlp_W1, mlp_b1.reshape(1, -1),
        mlp_W2, mlp_b2.reshape(1, -1), mlp_W3, mlp_b3.reshape(1, -1),
        fc_W, fc_b.reshape(1, -1))
